# baseline (speedup 1.0000x reference)
import jax
import jax.numpy as jnp
from jax.experimental import pallas as pl
from jax.experimental.pallas import tpu as pltpu


_N_DMA = 8


def _copy_body(w_ref, o_ref, sems):
    rows = w_ref.shape[0]
    chunk = rows // _N_DMA
    copies = [
        pltpu.make_async_copy(
            w_ref.at[pl.ds(i * chunk, chunk)],
            o_ref.at[pl.ds(i * chunk, chunk)],
            sems.at[i],
        )
        for i in range(_N_DMA)
    ]
    for c in copies:
        c.start()
    for c in copies:
        c.wait()


def kernel(x, emb_weight):
    rows, dim = emb_weight.shape
    assert x.shape[1] == rows and rows % _N_DMA == 0
    return pl.pallas_call(
        _copy_body,
        in_specs=[pl.BlockSpec(memory_space=pl.ANY)],
        out_specs=pl.BlockSpec(memory_space=pl.ANY),
        out_shape=jax.ShapeDtypeStruct((rows, dim), emb_weight.dtype),
        scratch_shapes=[pltpu.SemaphoreType.DMA((_N_DMA,))],
    )(emb_weight)


# speedup vs baseline: 30.2215x; 30.2215x over previous
import jax
import jax.numpy as jnp
from jax.experimental import pallas as pl


_BLOCK_ROWS = 256


def _copy_block(w_ref, o_ref):
    o_ref[...] = w_ref[...]


def kernel(x, emb_weight):
    rows, dim = emb_weight.shape
    assert x.shape[1] == rows and rows % _BLOCK_ROWS == 0
    grid = (rows // _BLOCK_ROWS,)
    return pl.pallas_call(
        _copy_block,
        grid=grid,
        in_specs=[pl.BlockSpec((_BLOCK_ROWS, dim), lambda i: (i, 0))],
        out_specs=pl.BlockSpec((_BLOCK_ROWS, dim), lambda i: (i, 0)),
        out_shape=jax.ShapeDtypeStruct((rows, dim), emb_weight.dtype),
    )(emb_weight)


# speedup vs baseline: 44.8756x; 1.4849x over previous
import jax
import jax.numpy as jnp
from jax.experimental import pallas as pl


_BLOCK_ROWS = 1024


def _copy_block(w_ref, o_ref):
    o_ref[...] = w_ref[...]


def kernel(x, emb_weight):
    rows, dim = emb_weight.shape
    assert x.shape[1] == rows and rows % _BLOCK_ROWS == 0
    grid = (rows // _BLOCK_ROWS,)
    return pl.pallas_call(
        _copy_block,
        grid=grid,
        in_specs=[pl.BlockSpec((_BLOCK_ROWS, dim), lambda i: (i, 0))],
        out_specs=pl.BlockSpec((_BLOCK_ROWS, dim), lambda i: (i, 0)),
        out_shape=jax.ShapeDtypeStruct((rows, dim), emb_weight.dtype),
    )(emb_weight)


# speedup vs baseline: 49.2924x; 1.0984x over previous
import jax
import jax.numpy as jnp
from jax.experimental import pallas as pl


_BLOCK_ROWS = 2048


def _copy_block(w_ref, o_ref):
    o_ref[...] = w_ref[...]


def kernel(x, emb_weight):
    rows, dim = emb_weight.shape
    assert x.shape[1] == rows and rows % _BLOCK_ROWS == 0
    grid = (rows // _BLOCK_ROWS,)
    return pl.pallas_call(
        _copy_block,
        grid=grid,
        in_specs=[pl.BlockSpec((_BLOCK_ROWS, dim), lambda i: (i, 0))],
        out_specs=pl.BlockSpec((_BLOCK_ROWS, dim), lambda i: (i, 0)),
        out_shape=jax.ShapeDtypeStruct((rows, dim), emb_weight.dtype),
    )(emb_weight)
